# baseline (speedup 1.0000x reference)
import jax
import jax.numpy as jnp
from jax.experimental import pallas as pl
from jax.experimental.pallas import tpu as pltpu

_SHIFTS = (0, 1, 3)
_COLS = 2048
_RB = 2


def _tok_kernel(x_ref, h_ref, o_ref):
    x = x_ref[...]
    h = h_ref[...]
    r = x.shape[0]
    for k, s in enumerate(_SHIFTS):
        o_ref[k] = jnp.concatenate([x[s:], h[:s]], axis=0) if s else x


def kernel(x_all):
    b, n = x_all.shape
    g = len(_SHIFTS)
    rows = n // _RB
    cols = min(_COLS, b)
    xt = x_all.T
    yt = pl.pallas_call(
        _tok_kernel,
        grid=(_RB, b // cols),
        in_specs=[
            pl.BlockSpec((rows, cols), lambda r, j: (r, j)),
            pl.BlockSpec((8, cols),
                         lambda r, j: (((r + 1) % _RB) * (rows // 8), j)),
        ],
        out_specs=pl.BlockSpec((g, rows, cols), lambda r, j: (0, r, j)),
        out_shape=jax.ShapeDtypeStruct((g, n, b), x_all.dtype),
        compiler_params=pltpu.CompilerParams(
            dimension_semantics=("arbitrary", "arbitrary"),
        ),
    )(xt, xt)
    return yt.transpose(2, 1, 0)
